# baseline (speedup 1.0000x reference)
import functools

import jax
import jax.numpy as jnp
from jax import lax
from jax.experimental import pallas as pl
from jax.experimental.pallas import tpu as pltpu
from jax.experimental.pallas import tpu_sc as plsc

NC = 2
NS = 16
NW = NC * NS
K = 128
DEGW = 128


def _acc_rows(n):
  per_tile = -(-(n + 1) // NS)
  per_tile = ((per_tile + 15) // 16) * 16
  return per_tile * NS


def _sc_degree(dst_blocks, ones_hbm, zeros_hbm, n):
  nw, c_chunks, k = dst_blocks.shape
  acc = _acc_rows(n)
  r = acc // NS
  mesh = plsc.VectorSubcoreMesh(core_axis_name="c", subcore_axis_name="s")

  @functools.partial(
      pl.kernel,
      mesh=mesh,
      out_type=jax.ShapeDtypeStruct((NC, acc, DEGW), jnp.float32),
      scratch_types=[
          pltpu.VMEM((c_chunks, k), jnp.int32),
          pltpu.VMEM((k, DEGW), jnp.float32),
          pltpu.VMEM_SHARED((acc, DEGW), jnp.float32),
      ],
  )
  def deg_kernel(dst_hbm, ones_in, zeros_in, out_hbm, dst_v, ones_v, deg_sh):
    c = lax.axis_index("c")
    s = lax.axis_index("s")
    wid = c * NS + s
    pltpu.sync_copy(dst_hbm.at[wid], dst_v)
    pltpu.sync_copy(ones_in, ones_v)
    pltpu.sync_copy(zeros_in, deg_sh.at[pl.ds(s * r, r)])
    plsc.subcore_barrier()

    def body(j, carry):
      pltpu.sync_copy(ones_v, deg_sh.at[dst_v.at[j]], add=True)
      return carry

    lax.fori_loop(0, c_chunks, body, 0)
    plsc.subcore_barrier()
    pltpu.sync_copy(deg_sh.at[pl.ds(s * r, r)], out_hbm.at[c, pl.ds(s * r, r)])

  return deg_kernel(dst_blocks, ones_hbm, zeros_hbm)


def _sc_scatter(src_blocks, dst_blocks, g, zeros_hbm, n):
  nw, c_chunks, k = src_blocks.shape
  d = g.shape[1]
  acc = _acc_rows(n)
  r = acc // NS
  mesh = plsc.VectorSubcoreMesh(core_axis_name="c", subcore_axis_name="s")

  assert c_chunks % 4 == 0
  half = c_chunks // 2

  @functools.partial(
      pl.kernel,
      mesh=mesh,
      out_type=jax.ShapeDtypeStruct((NC, acc, d), jnp.float32),
      scratch_types=[
          pltpu.VMEM((half, k), jnp.int32),
          pltpu.VMEM((half, k), jnp.int32),
          pltpu.VMEM((k, d), jnp.float32),
          pltpu.VMEM((k, d), jnp.float32),
          pltpu.VMEM_SHARED((acc, d), jnp.float32),
          pltpu.SemaphoreType.DMA,
      ],
  )
  def scat_kernel(src_hbm, dst_hbm, g_hbm, zeros_in, out_hbm,
                  src_v, dst_v, rows0, rows1, acc_sh, sem):
    c = lax.axis_index("c")
    s = lax.axis_index("s")
    wid = c * NS + s
    pltpu.sync_copy(zeros_in, acc_sh.at[pl.ds(s * r, r)])
    plsc.subcore_barrier()

    for ph in range(2):
      pltpu.sync_copy(src_hbm.at[wid, pl.ds(ph * half, half)], src_v)
      pltpu.sync_copy(dst_hbm.at[wid, pl.ds(ph * half, half)], dst_v)
      pltpu.async_copy(g_hbm.at[src_v.at[0]], rows0, sem).wait()

      def body(i, carry):
        j0 = 2 * i
        gd1 = pltpu.async_copy(g_hbm.at[src_v.at[j0 + 1]], rows1, sem)
        pltpu.sync_copy(rows0, acc_sh.at[dst_v.at[j0]], add=True)
        gd1.wait()
        gd2 = pltpu.async_copy(g_hbm.at[src_v.at[j0 + 2]], rows0, sem)
        pltpu.sync_copy(rows1, acc_sh.at[dst_v.at[j0 + 1]], add=True)
        gd2.wait()
        return carry

      lax.fori_loop(0, half // 2 - 1, body, 0)
      gd = pltpu.async_copy(g_hbm.at[src_v.at[half - 1]], rows1, sem)
      pltpu.sync_copy(rows0, acc_sh.at[dst_v.at[half - 2]], add=True)
      gd.wait()
      pltpu.sync_copy(rows1, acc_sh.at[dst_v.at[half - 1]], add=True)
    plsc.subcore_barrier()
    pltpu.sync_copy(acc_sh.at[pl.ds(s * r, r)], out_hbm.at[c, pl.ds(s * r, r)])

  return scat_kernel(src_blocks, dst_blocks, g, zeros_hbm)


_BLK = 1000


def _dis_from(d0, d1):
  deg = (d0[:, 0:1].astype(jnp.float32) + d1[:, 0:1].astype(jnp.float32)
         + 1.0)
  return lax.rsqrt(deg)


def _tc1a_body(x_ref, w_ref, o_ref):
  o_ref[...] = jnp.dot(x_ref[...], w_ref[...],
                       preferred_element_type=jnp.float32)


def _tc1b_body(h_ref, dp_ref0, dp_ref1, o_ref):
  dis = _dis_from(dp_ref0[0], dp_ref1[0])
  o_ref[...] = h_ref[...] * dis


def _tc2_body(g_ref, p0_ref, p1_ref, dp_ref0, dp_ref1, b_ref, w_ref, o_ref):
  dis = _dis_from(dp_ref0[0], dp_ref1[0])
  z = dis * (p0_ref[0] + p1_ref[0] + g_ref[...]) + b_ref[...]
  z = jnp.maximum(z, 0.0)
  h = jnp.dot(z, w_ref[...], preferred_element_type=jnp.float32)
  o_ref[...] = h * dis


def _tc3_body(g_ref, p0_ref, p1_ref, dp_ref0, dp_ref1, b_ref, o_ref):
  dis = _dis_from(dp_ref0[0], dp_ref1[0])
  o_ref[...] = dis * (p0_ref[0] + p1_ref[0] + g_ref[...]) + b_ref[...]


def _row_spec(w):
  return pl.BlockSpec((_BLK, w), lambda i: (i, 0))


def _part_spec(core, w):
  return pl.BlockSpec((1, _BLK, w), lambda i, c=core: (c, i, 0))


def _full_spec(h, w):
  return pl.BlockSpec((h, w), lambda i: (0, 0))


def _tc1a(x, w1):
  n, d = x.shape
  return pl.pallas_call(
      _tc1a_body,
      grid=(n // _BLK,),
      in_specs=[_row_spec(d), _full_spec(d, d)],
      out_specs=_row_spec(d),
      out_shape=jax.ShapeDtypeStruct((n, d), jnp.float32),
  )(x, w1)


def _tc1b(h, degp):
  n, d = h.shape
  return pl.pallas_call(
      _tc1b_body,
      grid=(n // _BLK,),
      in_specs=[_row_spec(d), _part_spec(0, DEGW), _part_spec(1, DEGW)],
      out_specs=_row_spec(d),
      out_shape=jax.ShapeDtypeStruct((n, d), jnp.float32),
  )(h, degp, degp)


def _tc2(g1, p, degp, b1, w2):
  n, d = g1.shape
  return pl.pallas_call(
      _tc2_body,
      grid=(n // _BLK,),
      in_specs=[_row_spec(d), _part_spec(0, d), _part_spec(1, d),
                _part_spec(0, DEGW), _part_spec(1, DEGW), _full_spec(1, d),
                _full_spec(d, d)],
      out_specs=_row_spec(d),
      out_shape=jax.ShapeDtypeStruct((n, d), jnp.float32),
  )(g1, p, p, degp, degp, b1, w2)


def _tc3(g2, q, degp, b2):
  n, d = g2.shape
  return pl.pallas_call(
      _tc3_body,
      grid=(n // _BLK,),
      in_specs=[_row_spec(d), _part_spec(0, d), _part_spec(1, d),
                _part_spec(0, DEGW), _part_spec(1, DEGW), _full_spec(1, d)],
      out_specs=_row_spec(d),
      out_shape=jax.ShapeDtypeStruct((n, d), jnp.float32),
  )(g2, q, q, degp, degp, b2)


def kernel(x, edge_index, W1, b1, W2, b2):
  n, d = x.shape
  e = edge_index.shape[1]
  acc = _acc_rows(n)
  r = acc // NS

  c_chunks = -(-e // (NW * K))
  c_chunks = ((c_chunks + 3) // 4) * 4
  e_pad = NW * c_chunks * K
  src = edge_index[0]
  dst = edge_index[1]
  pad = e_pad - e
  n_dummy = acc - n
  iot = jnp.arange(pad, dtype=jnp.int32)
  pad_src = jax.lax.rem(iot, jnp.int32(n))
  pad_dst = n + jax.lax.rem(iot, jnp.int32(n_dummy))
  src_b = jnp.concatenate([src, pad_src]).reshape(NW, c_chunks, K)
  dst_b = jnp.concatenate([dst, pad_dst]).reshape(NW, c_chunks, K)

  ones_kd = jnp.ones((K, DEGW), jnp.float32)
  zeros_rd = jnp.zeros((r, d), jnp.float32)

  degp = _sc_degree(dst_b, ones_kd, zeros_rd, n)

  h1 = _tc1a(x, W1)
  g1 = _tc1b(h1, degp)
  p = _sc_scatter(src_b, dst_b, g1, zeros_rd, n)
  g2 = _tc2(g1, p, degp, b1.reshape(1, d), W2)
  q = _sc_scatter(src_b, dst_b, g2, zeros_rd, n)
  out = _tc3(g2, q, degp, b2.reshape(1, d))
  return out


# speedup vs baseline: 1.0212x; 1.0212x over previous
import functools

import jax
import jax.numpy as jnp
from jax import lax
from jax.experimental import pallas as pl
from jax.experimental.pallas import tpu as pltpu
from jax.experimental.pallas import tpu_sc as plsc

NC = 2
NS = 16
NW = NC * NS
K = 128
DEGW = 128


def _acc_rows(n):
  per_tile = -(-(n + 1) // NS)
  per_tile = ((per_tile + 15) // 16) * 16
  return per_tile * NS


def _sc_degree(dst_blocks, ones_hbm, zeros_hbm, n):
  nw, c_chunks, k = dst_blocks.shape
  acc = _acc_rows(n)
  r = acc // NS
  mesh = plsc.VectorSubcoreMesh(core_axis_name="c", subcore_axis_name="s")

  @functools.partial(
      pl.kernel,
      mesh=mesh,
      out_type=jax.ShapeDtypeStruct((NC, acc, DEGW), jnp.float32),
      scratch_types=[
          pltpu.VMEM((c_chunks, k), jnp.int32),
          pltpu.VMEM((k, DEGW), jnp.float32),
          pltpu.VMEM_SHARED((acc, DEGW), jnp.float32),
      ],
  )
  def deg_kernel(dst_hbm, ones_in, zeros_in, out_hbm, dst_v, ones_v, deg_sh):
    c = lax.axis_index("c")
    s = lax.axis_index("s")
    wid = c * NS + s
    pltpu.sync_copy(dst_hbm.at[wid], dst_v)
    pltpu.sync_copy(ones_in, ones_v)
    pltpu.sync_copy(zeros_in, deg_sh.at[pl.ds(s * r, r)])
    plsc.subcore_barrier()

    def body(j, carry):
      pltpu.sync_copy(ones_v, deg_sh.at[dst_v.at[j]], add=True)
      return carry

    lax.fori_loop(0, c_chunks, body, 0)
    plsc.subcore_barrier()
    pltpu.sync_copy(deg_sh.at[pl.ds(s * r, r)], out_hbm.at[c, pl.ds(s * r, r)])

  return deg_kernel(dst_blocks, ones_hbm, zeros_hbm)


def _sc_scatter(src_blocks, dst_blocks, g, zeros_hbm, n):
  nw, c_chunks, k = src_blocks.shape
  d = g.shape[1]
  acc = _acc_rows(n)
  r = acc // NS
  mesh = plsc.VectorSubcoreMesh(core_axis_name="c", subcore_axis_name="s")

  assert c_chunks % 4 == 0
  half = c_chunks // 2

  @functools.partial(
      pl.kernel,
      mesh=mesh,
      out_type=jax.ShapeDtypeStruct((NC, acc, d), jnp.float32),
      scratch_types=[
          pltpu.VMEM((half, k), jnp.int32),
          pltpu.VMEM((half, k), jnp.int32),
          pltpu.VMEM((k, d), jnp.float32),
          pltpu.VMEM((k, d), jnp.float32),
          pltpu.VMEM_SHARED((acc, d), jnp.float32),
          pltpu.SemaphoreType.DMA,
          pltpu.SemaphoreType.DMA,
      ],
  )
  def scat_kernel(src_hbm, dst_hbm, g_hbm, zeros_in, out_hbm,
                  src_v, dst_v, rows0, rows1, acc_sh, sem, sem2):
    c = lax.axis_index("c")
    s = lax.axis_index("s")
    wid = c * NS + s
    kh = k // 2

    def gather(j, rows):
      ga = pltpu.async_copy(g_hbm.at[src_v.at[j, pl.ds(0, kh)]],
                            rows.at[pl.ds(0, kh)], sem)
      gb = pltpu.async_copy(g_hbm.at[src_v.at[j, pl.ds(kh, kh)]],
                            rows.at[pl.ds(kh, kh)], sem2)
      return ga, gb

    def wait(gds):
      gds[0].wait()
      gds[1].wait()

    pltpu.sync_copy(zeros_in, acc_sh.at[pl.ds(s * r, r)])
    plsc.subcore_barrier()

    for ph in range(2):
      pltpu.sync_copy(src_hbm.at[wid, pl.ds(ph * half, half)], src_v)
      pltpu.sync_copy(dst_hbm.at[wid, pl.ds(ph * half, half)], dst_v)
      wait(gather(0, rows0))

      def body(i, carry):
        j0 = 2 * i
        gd1 = gather(j0 + 1, rows1)
        pltpu.sync_copy(rows0, acc_sh.at[dst_v.at[j0]], add=True)
        wait(gd1)
        gd2 = gather(j0 + 2, rows0)
        pltpu.sync_copy(rows1, acc_sh.at[dst_v.at[j0 + 1]], add=True)
        wait(gd2)
        return carry

      lax.fori_loop(0, half // 2 - 1, body, 0)
      gd = gather(half - 1, rows1)
      pltpu.sync_copy(rows0, acc_sh.at[dst_v.at[half - 2]], add=True)
      wait(gd)
      pltpu.sync_copy(rows1, acc_sh.at[dst_v.at[half - 1]], add=True)
    plsc.subcore_barrier()
    pltpu.sync_copy(acc_sh.at[pl.ds(s * r, r)], out_hbm.at[c, pl.ds(s * r, r)])

  return scat_kernel(src_blocks, dst_blocks, g, zeros_hbm)


_BLK = 1000


def _dis_from(d0, d1):
  deg = (d0[:, 0:1].astype(jnp.float32) + d1[:, 0:1].astype(jnp.float32)
         + 1.0)
  return lax.rsqrt(deg)


def _tc1a_body(x_ref, w_ref, o_ref):
  o_ref[...] = jnp.dot(x_ref[...], w_ref[...],
                       preferred_element_type=jnp.float32)


def _tc1b_body(h_ref, dp_ref0, dp_ref1, o_ref):
  dis = _dis_from(dp_ref0[0], dp_ref1[0])
  o_ref[...] = h_ref[...] * dis


def _tc2_body(g_ref, p0_ref, p1_ref, dp_ref0, dp_ref1, b_ref, w_ref, o_ref):
  dis = _dis_from(dp_ref0[0], dp_ref1[0])
  z = dis * (p0_ref[0] + p1_ref[0] + g_ref[...]) + b_ref[...]
  z = jnp.maximum(z, 0.0)
  h = jnp.dot(z, w_ref[...], preferred_element_type=jnp.float32)
  o_ref[...] = h * dis


def _tc3_body(g_ref, p0_ref, p1_ref, dp_ref0, dp_ref1, b_ref, o_ref):
  dis = _dis_from(dp_ref0[0], dp_ref1[0])
  o_ref[...] = dis * (p0_ref[0] + p1_ref[0] + g_ref[...]) + b_ref[...]


def _row_spec(w):
  return pl.BlockSpec((_BLK, w), lambda i: (i, 0))


def _part_spec(core, w):
  return pl.BlockSpec((1, _BLK, w), lambda i, c=core: (c, i, 0))


def _full_spec(h, w):
  return pl.BlockSpec((h, w), lambda i: (0, 0))


def _tc1a(x, w1):
  n, d = x.shape
  return pl.pallas_call(
      _tc1a_body,
      grid=(n // _BLK,),
      in_specs=[_row_spec(d), _full_spec(d, d)],
      out_specs=_row_spec(d),
      out_shape=jax.ShapeDtypeStruct((n, d), jnp.float32),
  )(x, w1)


def _tc1b(h, degp):
  n, d = h.shape
  return pl.pallas_call(
      _tc1b_body,
      grid=(n // _BLK,),
      in_specs=[_row_spec(d), _part_spec(0, DEGW), _part_spec(1, DEGW)],
      out_specs=_row_spec(d),
      out_shape=jax.ShapeDtypeStruct((n, d), jnp.float32),
  )(h, degp, degp)


def _tc2(g1, p, degp, b1, w2):
  n, d = g1.shape
  return pl.pallas_call(
      _tc2_body,
      grid=(n // _BLK,),
      in_specs=[_row_spec(d), _part_spec(0, d), _part_spec(1, d),
                _part_spec(0, DEGW), _part_spec(1, DEGW), _full_spec(1, d),
                _full_spec(d, d)],
      out_specs=_row_spec(d),
      out_shape=jax.ShapeDtypeStruct((n, d), jnp.float32),
  )(g1, p, p, degp, degp, b1, w2)


def _tc3(g2, q, degp, b2):
  n, d = g2.shape
  return pl.pallas_call(
      _tc3_body,
      grid=(n // _BLK,),
      in_specs=[_row_spec(d), _part_spec(0, d), _part_spec(1, d),
                _part_spec(0, DEGW), _part_spec(1, DEGW), _full_spec(1, d)],
      out_specs=_row_spec(d),
      out_shape=jax.ShapeDtypeStruct((n, d), jnp.float32),
  )(g2, q, q, degp, degp, b2)


def kernel(x, edge_index, W1, b1, W2, b2):
  n, d = x.shape
  e = edge_index.shape[1]
  acc = _acc_rows(n)
  r = acc // NS

  c_chunks = -(-e // (NW * K))
  c_chunks = ((c_chunks + 3) // 4) * 4
  e_pad = NW * c_chunks * K
  src = edge_index[0]
  dst = edge_index[1]
  pad = e_pad - e
  n_dummy = acc - n
  iot = jnp.arange(pad, dtype=jnp.int32)
  pad_src = jax.lax.rem(iot, jnp.int32(n))
  pad_dst = n + jax.lax.rem(iot, jnp.int32(n_dummy))
  src_b = jnp.concatenate([src, pad_src]).reshape(NW, c_chunks, K)
  dst_b = jnp.concatenate([dst, pad_dst]).reshape(NW, c_chunks, K)

  ones_kd = jnp.ones((K, DEGW), jnp.float32)
  zeros_rd = jnp.zeros((r, d), jnp.float32)

  degp = _sc_degree(dst_b, ones_kd, zeros_rd, n)

  h1 = _tc1a(x, W1)
  g1 = _tc1b(h1, degp)
  p = _sc_scatter(src_b, dst_b, g1, zeros_rd, n)
  g2 = _tc2(g1, p, degp, b1.reshape(1, d), W2)
  q = _sc_scatter(src_b, dst_b, g2, zeros_rd, n)
  out = _tc3(g2, q, degp, b2.reshape(1, d))
  return out
